# Initial kernel scaffold; baseline (speedup 1.0000x reference)
import jax
import jax.numpy as jnp
from jax.experimental import pallas as pl


def kernel(enc_states, segments, group_count, W_ih, W_hh, b_ih, b_hh):
    raise NotImplementedError("write your pallas kernel here")


# speedup vs baseline: 4.8863x; 4.8863x over previous
import jax
import jax.numpy as jnp
from jax.experimental import pallas as pl
from jax.experimental.pallas import tpu as pltpu


def _pool_gi_kernel(seg_ref, enc_ref, wihT_ref, bih_ref, gi_ref):
    seg = seg_ref[0].astype(jnp.float32)
    denom = jnp.sum(seg, axis=1, keepdims=True) + 1.0
    alpha = seg / denom
    inps = jnp.dot(alpha, enc_ref[0], preferred_element_type=jnp.float32)
    gi = jnp.dot(inps, wihT_ref[...], preferred_element_type=jnp.float32)
    gi_ref[0] = gi + bih_ref[...]


def _gru_kernel(gi_ref, whhT_ref, bhh_ref, gc_ref, out_ref):
    B, G, H3 = gi_ref.shape
    H = H3 // 3
    gc = gc_ref[...]

    def step(t, h):
        gi = gi_ref[:, t, :]
        gh = jnp.dot(h, whhT_ref[...],
                     preferred_element_type=jnp.float32) + bhh_ref[...]
        r = jax.nn.sigmoid(gi[:, :H] + gh[:, :H])
        z = jax.nn.sigmoid(gi[:, H:2 * H] + gh[:, H:2 * H])
        n = jnp.tanh(gi[:, 2 * H:] + r * gh[:, 2 * H:])
        h_new = (1.0 - z) * n + z * h
        mask = t < gc
        out_ref[:, t, :] = jnp.where(mask, h_new, 0.0)
        return jnp.where(mask, h_new, h)

    jax.lax.fori_loop(0, G, step, jnp.zeros((B, H), jnp.float32))


def kernel(enc_states, segments, group_count, W_ih, W_hh, b_ih, b_hh):
    B, S, D = enc_states.shape
    G = segments.shape[1]
    H = W_hh.shape[1]

    wihT = W_ih.T
    whhT = W_hh.T
    bih = b_ih.reshape(1, 3 * H)
    bhh = b_hh.reshape(1, 3 * H)
    gc = group_count.reshape(B, 1)

    gi = pl.pallas_call(
        _pool_gi_kernel,
        grid=(B,),
        in_specs=[
            pl.BlockSpec((1, G, S), lambda b: (b, 0, 0)),
            pl.BlockSpec((1, S, D), lambda b: (b, 0, 0)),
            pl.BlockSpec((D, 3 * H), lambda b: (0, 0)),
            pl.BlockSpec((1, 3 * H), lambda b: (0, 0)),
        ],
        out_specs=pl.BlockSpec((1, G, 3 * H), lambda b: (b, 0, 0)),
        out_shape=jax.ShapeDtypeStruct((B, G, 3 * H), jnp.float32),
        compiler_params=pltpu.CompilerParams(
            dimension_semantics=("arbitrary",)),
    )(segments, enc_states, wihT, bih)

    out = pl.pallas_call(
        _gru_kernel,
        out_shape=jax.ShapeDtypeStruct((B, G, H), jnp.float32),
    )(gi, whhT, bhh, gc)
    return out


# speedup vs baseline: 4.9629x; 1.0157x over previous
import jax
import jax.numpy as jnp
from jax.experimental import pallas as pl
from jax.experimental.pallas import tpu as pltpu


def _contract_last(x, w):
    return jax.lax.dot_general(
        x, w, dimension_numbers=(((1,), (1,)), ((), ())),
        preferred_element_type=jnp.float32)


def _fused_kernel(seg_ref, enc_ref, wih_ref, whh_ref, bih_ref, bhh_ref,
                  gc_ref, out_ref, gi_ref, h_ref):
    B, G, H3 = gi_ref.shape
    H = H3 // 3
    i = pl.program_id(0)

    @pl.when(i < B)
    def _phase_a():
        seg = seg_ref[0].astype(jnp.float32)
        denom = jnp.sum(seg, axis=1, keepdims=True) + 1.0
        alpha = seg / denom
        inps = jnp.dot(alpha, enc_ref[0],
                       preferred_element_type=jnp.float32)
        gi = _contract_last(inps.astype(jnp.bfloat16), wih_ref[...])
        gi_ref[i] = gi + bih_ref[...]

    @pl.when(i == B)
    def _init_h():
        h_ref[...] = jnp.zeros_like(h_ref)

    @pl.when(i >= B)
    def _phase_b():
        t = i - B
        h = h_ref[...]
        gi = gi_ref[:, t, :]
        gh = _contract_last(h.astype(jnp.bfloat16),
                            whh_ref[...]) + bhh_ref[...]
        r = jax.nn.sigmoid(gi[:, :H] + gh[:, :H])
        z = jax.nn.sigmoid(gi[:, H:2 * H] + gh[:, H:2 * H])
        n = jnp.tanh(gi[:, 2 * H:] + r * gh[:, 2 * H:])
        h_new = (1.0 - z) * n + z * h
        mask = t < gc_ref[...]
        out_ref[:, t, :] = jnp.where(mask, h_new, 0.0)
        h_ref[...] = jnp.where(mask, h_new, h)


def kernel(enc_states, segments, group_count, W_ih, W_hh, b_ih, b_hh):
    B, S, D = enc_states.shape
    G = segments.shape[1]
    H = W_hh.shape[1]

    wih = W_ih.astype(jnp.bfloat16)
    whh = W_hh.astype(jnp.bfloat16)
    bih = b_ih.reshape(1, 3 * H)
    bhh = b_hh.reshape(1, 3 * H)
    gc = group_count.reshape(B, 1)

    last_a = B - 1
    out = pl.pallas_call(
        _fused_kernel,
        grid=(B + G,),
        in_specs=[
            pl.BlockSpec((1, G, S), lambda i: (jnp.minimum(i, last_a), 0, 0)),
            pl.BlockSpec((1, S, D), lambda i: (jnp.minimum(i, last_a), 0, 0)),
            pl.BlockSpec((3 * H, D), lambda i: (0, 0)),
            pl.BlockSpec((3 * H, H), lambda i: (0, 0)),
            pl.BlockSpec((1, 3 * H), lambda i: (0, 0)),
            pl.BlockSpec((1, 3 * H), lambda i: (0, 0)),
            pl.BlockSpec((B, 1), lambda i: (0, 0)),
        ],
        out_specs=pl.BlockSpec((B, G, H), lambda i: (0, 0, 0)),
        out_shape=jax.ShapeDtypeStruct((B, G, H), jnp.float32),
        scratch_shapes=[
            pltpu.VMEM((B, G, 3 * H), jnp.float32),
            pltpu.VMEM((B, H), jnp.float32),
        ],
        compiler_params=pltpu.CompilerParams(
            dimension_semantics=("arbitrary",)),
    )(segments, enc_states, wih, whh, bih, bhh, gc)
    return out
